# baseline (speedup 1.0000x reference)
import functools

import jax
import jax.numpy as jnp
from jax import lax
from jax.experimental import pallas as pl
from jax.experimental.pallas import tpu as pltpu
from jax.experimental.pallas import tpu_sc as plsc

_K = 8192
_D = 32
_N = 8192
_TILE = 1024
_BETA = 0.25

_NC = 2
_NS = 16
_NW = _NC * _NS
_BPW = _N // _NW
_GCH = 128
_CPW = _BPW // _GCH
_DPAD = 128


def _dist_argmin_body(z2_ref, e2_ref, z_ref, emb_ref, idx_ref, dsum_ref):
    mm = lax.dot_general(z_ref[...], emb_ref[...],
                         dimension_numbers=(((1,), (1,)), ((), ())))
    d = (z2_ref[...] + e2_ref[...]) - 2.0 * mm
    dmin = jnp.min(d, axis=1, keepdims=True)
    kiota = lax.broadcasted_iota(jnp.int32, (_TILE, _K), 1)
    idx = jnp.min(jnp.where(d == dmin, kiota, _K), axis=1)
    i = pl.program_id(0)
    idx_ref[i // (1024 // _TILE), pl.ds((i % (1024 // _TILE)) * _TILE, _TILE)] = idx

    @pl.when(i == 0)
    def _init():
        dsum_ref[0, 0] = 0.0

    dsum_ref[0, 0] += jnp.sum(dmin)


def _make_dist_call(rows):
    return pl.pallas_call(
        _dist_argmin_body,
        grid=(rows // _TILE,),
        in_specs=[
            pl.BlockSpec((_TILE, 1), lambda i: (i, 0)),
            pl.BlockSpec((1, _K), lambda i: (0, 0)),
            pl.BlockSpec((_TILE, _D), lambda i: (i, 0)),
            pl.BlockSpec((_K, _D), lambda i: (0, 0)),
        ],
        out_specs=[
            pl.BlockSpec((rows // 1024, 1024), lambda i: (0, 0)),
            pl.BlockSpec((1, 1), lambda i: (0, 0), memory_space=pltpu.SMEM),
        ],
        out_shape=[
            jax.ShapeDtypeStruct((rows // 1024, 1024), jnp.int32),
            jax.ShapeDtypeStruct((1, 1), jnp.float32),
        ],
    )


_dist_call_half = _make_dist_call(_N // 2)


@functools.cache
def _build_gather_kernel(rows):
    bpw = rows // _NW
    cpw = max(1, bpw // _GCH)
    gch = min(bpw, _GCH)
    wpr = 1024 // bpw

    @functools.partial(
        pl.kernel,
        mesh=plsc.VectorSubcoreMesh(core_axis_name="c", subcore_axis_name="s"),
        out_type=jax.ShapeDtypeStruct((rows, _DPAD), jnp.float32),
        scratch_types=[
            pltpu.VMEM((cpw, gch), jnp.int32),
            pltpu.VMEM((bpw, _DPAD), jnp.float32),
            pltpu.SemaphoreType.DMA,
        ],
    )
    def _gather_kernel(idx_hbm, table_hbm, out_hbm, idx_v, rows_v, sem):
        wid = lax.axis_index("s") * _NC + lax.axis_index("c")
        row = wid // wpr
        col = (wid % wpr) * bpw
        for j in range(cpw):
            pltpu.sync_copy(idx_hbm.at[row, pl.ds(col + j * gch, gch)],
                            idx_v.at[j])
        copies = [
            pltpu.async_copy(table_hbm.at[idx_v.at[j]],
                             rows_v.at[pl.ds(j * gch, gch)], sem)
            for j in range(cpw)
        ]
        for c in copies:
            c.wait()
        pltpu.sync_copy(rows_v, out_hbm.at[pl.ds(wid * bpw, bpw)])

    return _gather_kernel


def _st_transpose_body(z_ref, q_ref, out_ref):
    q = q_ref[:, 0:_D]
    st = z_ref[...] + (q - z_ref[...])
    out_ref[0] = st.T.reshape(_D, 32, 32)


def _st_transpose_body_aliased(prev_ref, z_ref, q_ref, out_ref):
    del prev_ref
    _st_transpose_body(z_ref, q_ref, out_ref)


_st_callA = pl.pallas_call(
    _st_transpose_body,
    grid=(4,),
    in_specs=[
        pl.BlockSpec((1024, _D), lambda i: (i, 0)),
        pl.BlockSpec((1024, _DPAD), lambda i: (i, 0)),
    ],
    out_specs=pl.BlockSpec((1, _D, 32, 32), lambda i: (i, 0, 0, 0)),
    out_shape=jax.ShapeDtypeStruct((8, _D, 32, 32), jnp.float32),
)

_st_callB = pl.pallas_call(
    _st_transpose_body_aliased,
    grid=(4,),
    in_specs=[
        pl.BlockSpec(memory_space=pl.ANY),
        pl.BlockSpec((1024, _D), lambda i: (i + 4, 0)),
        pl.BlockSpec((1024, _DPAD), lambda i: (i, 0)),
    ],
    out_specs=pl.BlockSpec((1, _D, 32, 32), lambda i: (i + 4, 0, 0, 0)),
    out_shape=jax.ShapeDtypeStruct((8, _D, 32, 32), jnp.float32),
    input_output_aliases={0: 0},
)


def kernel(z, embedding):
    b, c, h, w = z.shape
    zf = jnp.transpose(z, (0, 2, 3, 1)).reshape(-1, _D)
    z2 = jnp.sum(zf ** 2, axis=1, keepdims=True)
    e2 = jnp.sum(embedding ** 2, axis=1).reshape(1, _K)
    half = _N // 2
    idx_a, dsum_a = _dist_call_half(z2[:half], e2, zf[:half], embedding)
    idx_b, dsum_b = _dist_call_half(z2[half:], e2, zf[half:], embedding)

    emb_pad = jnp.pad(embedding, ((0, 0), (0, _DPAD - _D)))
    gather = _build_gather_kernel(half)
    q_a = gather(idx_a, emb_pad)
    q_b = gather(idx_b, emb_pad)

    out_a = _st_callA(zf, q_a)
    out = _st_callB(out_a, zf, q_b)

    dsum = dsum_a[0, 0] + dsum_b[0, 0]
    mean_min_dist = dsum / (_N * _D)
    loss = mean_min_dist + _BETA * mean_min_dist
    return (out, loss, jnp.concatenate([idx_a, idx_b], axis=0))


# speedup vs baseline: 1.1087x; 1.1087x over previous
import functools

import jax
import jax.numpy as jnp
from jax import lax
from jax.experimental import pallas as pl
from jax.experimental.pallas import tpu as pltpu
from jax.experimental.pallas import tpu_sc as plsc

_K = 8192
_D = 32
_N = 8192
_TILE = 1024
_BETA = 0.25

_NC = 2
_NS = 16
_NW = _NC * _NS
_BPW = _N // _NW
_GCH = 128
_CPW = _BPW // _GCH
_DPAD = 128


def _dist_argmin_body(z2_ref, e2_ref, z_ref, emb_ref, idx_ref, dsum_ref):
    mm = lax.dot_general(z_ref[...], emb_ref[...],
                         dimension_numbers=(((1,), (1,)), ((), ())))
    d = (z2_ref[...] + e2_ref[...]) - 2.0 * mm
    dmin = jnp.min(d, axis=1, keepdims=True)
    kiota = lax.broadcasted_iota(jnp.int32, (_TILE, _K), 1)
    idx = jnp.min(jnp.where(d == dmin, kiota, _K), axis=1)
    i = pl.program_id(0)
    idx_ref[i // (1024 // _TILE), pl.ds((i % (1024 // _TILE)) * _TILE, _TILE)] = idx

    @pl.when(i == 0)
    def _init():
        dsum_ref[0, 0] = 0.0

    dsum_ref[0, 0] += jnp.sum(dmin)


_dist_call = pl.pallas_call(
    _dist_argmin_body,
    grid=(_N // _TILE,),
    in_specs=[
        pl.BlockSpec((_TILE, 1), lambda i: (i, 0)),
        pl.BlockSpec((1, _K), lambda i: (0, 0)),
        pl.BlockSpec((_TILE, _D), lambda i: (i, 0)),
        pl.BlockSpec((_K, _D), lambda i: (0, 0)),
    ],
    out_specs=[
        pl.BlockSpec((8, 1024), lambda i: (0, 0)),
        pl.BlockSpec((1, 1), lambda i: (0, 0), memory_space=pltpu.SMEM),
    ],
    out_shape=[
        jax.ShapeDtypeStruct((8, 1024), jnp.int32),
        jax.ShapeDtypeStruct((1, 1), jnp.float32),
    ],
)


@functools.cache
def _build_gather_kernel():
    @functools.partial(
        pl.kernel,
        mesh=plsc.VectorSubcoreMesh(core_axis_name="c", subcore_axis_name="s"),
        out_type=jax.ShapeDtypeStruct((_N, _DPAD), jnp.float32),
        scratch_types=[
            pltpu.VMEM((_CPW, _GCH), jnp.int32),
            pltpu.VMEM((_BPW, _DPAD), jnp.float32),
            pltpu.SemaphoreType.DMA,
        ],
    )
    def _gather_kernel(idx_hbm, table_hbm, out_hbm, idx_v, rows_v, sem):
        wid = lax.axis_index("s") * _NC + lax.axis_index("c")
        row = wid // 4
        col = (wid % 4) * _BPW
        for j in range(_CPW):
            pltpu.sync_copy(idx_hbm.at[row, pl.ds(col + j * _GCH, _GCH)],
                            idx_v.at[j])
        copies = [
            pltpu.async_copy(table_hbm.at[idx_v.at[j]],
                             rows_v.at[pl.ds(j * _GCH, _GCH)], sem)
            for j in range(_CPW)
        ]
        for c in copies:
            c.wait()
        pltpu.sync_copy(rows_v, out_hbm.at[pl.ds(wid * _BPW, _BPW)])

    return _gather_kernel


def _st_transpose_body(z_ref, q_ref, out_ref):
    q = q_ref[:, 0:_D]
    st = z_ref[...] + (q - z_ref[...])
    out_ref[0] = st.T.reshape(_D, 32, 32)


_st_call = pl.pallas_call(
    _st_transpose_body,
    grid=(8,),
    in_specs=[
        pl.BlockSpec((_N // 8, _D), lambda i: (i, 0)),
        pl.BlockSpec((_N // 8, _DPAD), lambda i: (i, 0)),
    ],
    out_specs=pl.BlockSpec((1, _D, 32, 32), lambda i: (i, 0, 0, 0)),
    out_shape=jax.ShapeDtypeStruct((8, _D, 32, 32), jnp.float32),
)


def kernel(z, embedding):
    b, c, h, w = z.shape
    zf = jnp.transpose(z, (0, 2, 3, 1)).reshape(-1, _D)
    z2 = jnp.sum(zf ** 2, axis=1, keepdims=True)
    e2 = jnp.sum(embedding ** 2, axis=1).reshape(1, _K)
    idx_out, dsum = _dist_call(z2, e2, zf, embedding)

    emb_pad = jnp.pad(embedding, ((0, 0), (0, _DPAD - _D)))
    q = _build_gather_kernel()(idx_out, emb_pad)

    out = _st_call(zf, q)

    mean_min_dist = dsum[0, 0] / (_N * _D)
    loss = mean_min_dist + _BETA * mean_min_dist
    return (out, loss, idx_out)
